# baseline (speedup 1.0000x reference)
import jax
import jax.numpy as jnp
from jax import lax
from jax.experimental import pallas as pl
from jax.experimental.pallas import tpu as pltpu
from jax.experimental.pallas import tpu_sc as plsc

_NC = 100
_B = 65536
_NCORES = 2
_NSUB = 16
_NW = _NCORES * _NSUB
_RPW = _B // _NW
_CHR = 256
_NCHUNK = _RPW // _CHR


def _sc_body(x_hbm, t_hbm, w_hbm, out_hbm,
             xb0, xb1, trow, wv, binsum, bincnt, outv,
             s0, s1, st, sw):
    wid = lax.axis_index("s") * _NCORES + lax.axis_index("c")
    base = wid * _RPW
    lane = lax.iota(jnp.int32, 16)
    lane112 = lane * 112
    zero16 = jnp.zeros((16,), jnp.float32)
    one16 = jnp.ones((16,), jnp.float32)

    tcp = pltpu.async_copy(t_hbm.at[pl.ds(base, _RPW)], trow, st)
    wv[pl.ds(96, 16)] = zero16
    wcp = pltpu.async_copy(w_hbm.at[pl.ds(0, _NC)], wv.at[pl.ds(0, _NC)], sw)

    def xsrc(ci):
        return x_hbm.at[pl.ds(base + ci * _CHR, _CHR), :]

    pltpu.async_copy(xsrc(0), xb0, s0)
    pltpu.async_copy(xsrc(1), xb1, s1)

    for l in range(16):
        for j in range(7):
            binsum[pl.ds(l * 112 + j * 16, 16)] = zero16
            bincnt[pl.ds(l * 112 + j * 16, 16)] = zero16

    tcp.wait()
    wcp.wait()
    wvecs = [wv[pl.ds(j * 16, 16)] for j in range(7)]

    def process(xb, ci):
        def gbody(g, carry):
            rowidx = lane + g * 16
            sacc = zero16
            for c in range(_NC):
                col = plsc.load_gather(
                    xb, [rowidx, jnp.full((16,), c, jnp.int32)])
                sacc = sacc + jnp.exp(col) * wvecs[c // 16][c % 16]
            tg = trow[pl.ds(ci * _CHR + g * 16, 16)]
            xt = plsc.load_gather(xb, [rowidx, tg])
            wt = plsc.load_gather(wv, [tg])
            et = jnp.exp(xt) * wt
            pr = jnp.minimum(jnp.maximum(et / sacc, 1e-5), 1.0)
            flat = lane112 + tg
            plsc.addupdate_scatter(binsum, [flat], pr)
            plsc.addupdate_scatter(bincnt, [flat], one16)
            return carry

        lax.fori_loop(0, _CHR // 16, gbody, 0)

    def chunk_body(ci, carry):
        @pl.when(ci % 2 == 0)
        def _():
            pltpu.make_async_copy(xsrc(ci), xb0, s0).wait()
            process(xb0, ci)

            @pl.when(ci + 2 < _NCHUNK)
            def _():
                pltpu.async_copy(xsrc(ci + 2), xb0, s0)

        @pl.when(ci % 2 == 1)
        def _():
            pltpu.make_async_copy(xsrc(ci), xb1, s1).wait()
            process(xb1, ci)

            @pl.when(ci + 2 < _NCHUNK)
            def _():
                pltpu.async_copy(xsrc(ci + 2), xb1, s1)

        return carry

    lax.fori_loop(0, _NCHUNK, chunk_body, 0)

    for j in range(7):
        accs = zero16
        accc = zero16
        for l in range(16):
            accs = accs + binsum[pl.ds(l * 112 + j * 16, 16)]
            accc = accc + bincnt[pl.ds(l * 112 + j * 16, 16)]
        outv[0, pl.ds(j * 16, 16)] = accs
        outv[1, pl.ds(j * 16, 16)] = accc
    outv[0, pl.ds(112, 16)] = zero16
    outv[1, pl.ds(112, 16)] = zero16

    pltpu.sync_copy(outv.at[0], out_hbm.at[wid])
    pltpu.sync_copy(outv.at[1], out_hbm.at[_NW + wid])


def _fin_body(pref, oref):
    sums = jnp.sum(pref[0:_NW, :], axis=0, keepdims=True)
    counts = jnp.sum(pref[_NW:2 * _NW, :], axis=0, keepdims=True)
    exist = counts != 0.0
    denom = jnp.where(exist, counts, 1.0)
    meanp = sums / denom
    safe = jnp.where(exist, meanp, 1.0)
    ml = -jnp.log(safe)
    pw = jnp.where(exist, ml * ml * ml, 0.0)
    n_exist = jnp.sum(exist.astype(jnp.float32))
    msum = jnp.sum(pw) / n_exist
    loss = jnp.exp(jnp.log(msum) / 3.0)
    oref[...] = jnp.broadcast_to(loss, (1, 1))


def kernel(output, target, weight):
    mesh = plsc.VectorSubcoreMesh(core_axis_name="c", subcore_axis_name="s",
                                  num_cores=_NCORES, num_subcores=_NSUB)
    sc = pl.kernel(
        _sc_body,
        out_type=jax.ShapeDtypeStruct((2 * _NW, 128), jnp.float32),
        mesh=mesh,
        compiler_params=pltpu.CompilerParams(needs_layout_passes=False),
        scratch_types=[
            pltpu.VMEM((_CHR, _NC), jnp.float32),
            pltpu.VMEM((_CHR, _NC), jnp.float32),
            pltpu.VMEM((_RPW,), jnp.int32),
            pltpu.VMEM((112,), jnp.float32),
            pltpu.VMEM((1792,), jnp.float32),
            pltpu.VMEM((1792,), jnp.float32),
            pltpu.VMEM((2, 128), jnp.float32),
            pltpu.SemaphoreType.DMA,
            pltpu.SemaphoreType.DMA,
            pltpu.SemaphoreType.DMA,
            pltpu.SemaphoreType.DMA,
        ],
    )
    partials = sc(output, target, weight)
    res = pl.pallas_call(
        _fin_body,
        out_shape=jax.ShapeDtypeStruct((1, 1), jnp.float32),
    )(partials)
    return res[0, 0]


# speedup vs baseline: 2.0105x; 2.0105x over previous
import jax
import jax.numpy as jnp
from jax import lax
from jax.experimental import pallas as pl
from jax.experimental.pallas import tpu as pltpu
from jax.experimental.pallas import tpu_sc as plsc

_NC = 100
_B = 65536
_NCORES = 2
_NSUB = 16
_NW = _NCORES * _NSUB
_RPW = _B // _NW
_CHR = 256
_NCHUNK = _RPW // _CHR


def _sc_body(x_hbm, t_hbm, w_hbm, out_hbm,
             xb0, xb1, trow, wv, wrot, binsum, bincnt, outv,
             s0, s1, st, sw):
    wid = lax.axis_index("s") * _NCORES + lax.axis_index("c")
    base = wid * _RPW
    lane = lax.iota(jnp.int32, 16)
    lane112 = lane * 112
    zero16 = jnp.zeros((16,), jnp.float32)
    one16 = jnp.ones((16,), jnp.float32)

    tcp = pltpu.async_copy(t_hbm.at[pl.ds(base, _RPW)], trow, st)
    wv[pl.ds(96, 16)] = zero16
    wcp = pltpu.async_copy(w_hbm.at[pl.ds(0, _NC)], wv.at[pl.ds(0, _NC)], sw)

    def xsrc(ci):
        return x_hbm.at[pl.ds(base + ci * _CHR, _CHR), :]

    pltpu.async_copy(xsrc(0), xb0, s0)
    pltpu.async_copy(xsrc(1), xb1, s1)

    for l in range(16):
        for j in range(7):
            binsum[pl.ds(l * 112 + j * 16, 16)] = zero16
            bincnt[pl.ds(l * 112 + j * 16, 16)] = zero16

    tcp.wait()
    wcp.wait()
    wvecs = [wv[pl.ds(j * 16, 16)] for j in range(7)]

    def wrot_body(c, carry):
        cv = lane + c
        cv = jnp.where(cv >= _NC, cv - _NC, cv)
        wrot[pl.ds(c * 16, 16)] = plsc.load_gather(wv, [cv])
        return carry

    lax.fori_loop(0, _NC, wrot_body, 0)

    def process(xb, ci):
        def gbody(g, carry):
            rowidx = lane + g * 16
            saccs = [zero16, zero16, zero16, zero16]
            for c in range(_NC):
                cv = lane + c
                cv = jnp.where(cv >= _NC, cv - _NC, cv)
                col = plsc.load_gather(xb, [rowidx, cv])
                wc = wrot[pl.ds(c * 16, 16)]
                saccs[c % 4] = saccs[c % 4] + jnp.exp(col) * wc
            sacc = (saccs[0] + saccs[1]) + (saccs[2] + saccs[3])
            tg = trow[pl.ds(ci * _CHR + g * 16, 16)]
            xt = plsc.load_gather(xb, [rowidx, tg])
            wt = plsc.load_gather(wv, [tg])
            et = jnp.exp(xt) * wt
            pr = jnp.minimum(jnp.maximum(et / sacc, 1e-5), 1.0)
            flat = lane112 + tg
            plsc.addupdate_scatter(binsum, [flat], pr)
            plsc.addupdate_scatter(bincnt, [flat], one16)
            return carry

        lax.fori_loop(0, _CHR // 16, gbody, 0)

    def chunk_body(ci, carry):
        @pl.when(ci % 2 == 0)
        def _():
            pltpu.make_async_copy(xsrc(ci), xb0, s0).wait()
            process(xb0, ci)

            @pl.when(ci + 2 < _NCHUNK)
            def _():
                pltpu.async_copy(xsrc(ci + 2), xb0, s0)

        @pl.when(ci % 2 == 1)
        def _():
            pltpu.make_async_copy(xsrc(ci), xb1, s1).wait()
            process(xb1, ci)

            @pl.when(ci + 2 < _NCHUNK)
            def _():
                pltpu.async_copy(xsrc(ci + 2), xb1, s1)

        return carry

    lax.fori_loop(0, _NCHUNK, chunk_body, 0)

    for j in range(7):
        accs = zero16
        accc = zero16
        for l in range(16):
            accs = accs + binsum[pl.ds(l * 112 + j * 16, 16)]
            accc = accc + bincnt[pl.ds(l * 112 + j * 16, 16)]
        outv[0, pl.ds(j * 16, 16)] = accs
        outv[1, pl.ds(j * 16, 16)] = accc
    outv[0, pl.ds(112, 16)] = zero16
    outv[1, pl.ds(112, 16)] = zero16

    pltpu.sync_copy(outv.at[0], out_hbm.at[wid])
    pltpu.sync_copy(outv.at[1], out_hbm.at[_NW + wid])


def _fin_body(pref, oref):
    sums = jnp.sum(pref[0:_NW, :], axis=0, keepdims=True)
    counts = jnp.sum(pref[_NW:2 * _NW, :], axis=0, keepdims=True)
    exist = counts != 0.0
    denom = jnp.where(exist, counts, 1.0)
    meanp = sums / denom
    safe = jnp.where(exist, meanp, 1.0)
    ml = -jnp.log(safe)
    pw = jnp.where(exist, ml * ml * ml, 0.0)
    n_exist = jnp.sum(exist.astype(jnp.float32))
    msum = jnp.sum(pw) / n_exist
    loss = jnp.exp(jnp.log(msum) / 3.0)
    oref[...] = jnp.broadcast_to(loss, (1, 1))


def kernel(output, target, weight):
    mesh = plsc.VectorSubcoreMesh(core_axis_name="c", subcore_axis_name="s",
                                  num_cores=_NCORES, num_subcores=_NSUB)
    sc = pl.kernel(
        _sc_body,
        out_type=jax.ShapeDtypeStruct((2 * _NW, 128), jnp.float32),
        mesh=mesh,
        compiler_params=pltpu.CompilerParams(needs_layout_passes=False),
        scratch_types=[
            pltpu.VMEM((_CHR, _NC), jnp.float32),
            pltpu.VMEM((_CHR, _NC), jnp.float32),
            pltpu.VMEM((_RPW,), jnp.int32),
            pltpu.VMEM((112,), jnp.float32),
            pltpu.VMEM((1600,), jnp.float32),
            pltpu.VMEM((1792,), jnp.float32),
            pltpu.VMEM((1792,), jnp.float32),
            pltpu.VMEM((2, 128), jnp.float32),
            pltpu.SemaphoreType.DMA,
            pltpu.SemaphoreType.DMA,
            pltpu.SemaphoreType.DMA,
            pltpu.SemaphoreType.DMA,
        ],
    )
    partials = sc(output, target, weight)
    res = pl.pallas_call(
        _fin_body,
        out_shape=jax.ShapeDtypeStruct((1, 1), jnp.float32),
    )(partials)
    return res[0, 0]


# speedup vs baseline: 2.1749x; 1.0818x over previous
import jax
import jax.numpy as jnp
from jax import lax
from jax.experimental import pallas as pl
from jax.experimental.pallas import tpu as pltpu
from jax.experimental.pallas import tpu_sc as plsc

_NC = 100
_B = 65536
_NCORES = 2
_NSUB = 16
_NW = _NCORES * _NSUB
_BSC = _B // 2
_RPW = _BSC // _NW
_CHR = 256
_NCHUNK = _RPW // _CHR


def _sc_body(x_hbm, t_hbm, w_hbm, out_hbm,
             xb0, xb1, trow, wv, wrot, binsum, bincnt, outv,
             s0, s1, st, sw):
    wid = lax.axis_index("s") * _NCORES + lax.axis_index("c")
    base = wid * _RPW
    lane = lax.iota(jnp.int32, 16)
    lane112 = lane * 112
    zero16 = jnp.zeros((16,), jnp.float32)
    one16 = jnp.ones((16,), jnp.float32)

    tcp = pltpu.async_copy(t_hbm.at[pl.ds(base, _RPW)], trow, st)
    wv[pl.ds(96, 16)] = zero16
    wcp = pltpu.async_copy(w_hbm.at[pl.ds(0, _NC)], wv.at[pl.ds(0, _NC)], sw)

    def xsrc(ci):
        return x_hbm.at[pl.ds(base + ci * _CHR, _CHR), :]

    pltpu.async_copy(xsrc(0), xb0, s0)
    pltpu.async_copy(xsrc(1), xb1, s1)

    for l in range(16):
        for j in range(7):
            binsum[pl.ds(l * 112 + j * 16, 16)] = zero16
            bincnt[pl.ds(l * 112 + j * 16, 16)] = zero16

    tcp.wait()
    wcp.wait()
    wvecs = [wv[pl.ds(j * 16, 16)] for j in range(7)]

    def wrot_body(c, carry):
        cv = lane + c
        cv = jnp.where(cv >= _NC, cv - _NC, cv)
        wrot[pl.ds(c * 16, 16)] = plsc.load_gather(wv, [cv])
        return carry

    lax.fori_loop(0, _NC, wrot_body, 0)

    def process(xb, ci):
        def gbody(g, carry):
            rowidx = lane + g * 16
            saccs = [zero16, zero16, zero16, zero16]
            for c in range(_NC):
                cv = lane + c
                cv = jnp.where(cv >= _NC, cv - _NC, cv)
                col = plsc.load_gather(xb, [rowidx, cv])
                wc = wrot[pl.ds(c * 16, 16)]
                saccs[c % 4] = saccs[c % 4] + jnp.exp(col) * wc
            sacc = (saccs[0] + saccs[1]) + (saccs[2] + saccs[3])
            tg = trow[pl.ds(ci * _CHR + g * 16, 16)]
            xt = plsc.load_gather(xb, [rowidx, tg])
            wt = plsc.load_gather(wv, [tg])
            et = jnp.exp(xt) * wt
            pr = jnp.minimum(jnp.maximum(et / sacc, 1e-5), 1.0)
            flat = lane112 + tg
            plsc.addupdate_scatter(binsum, [flat], pr)
            plsc.addupdate_scatter(bincnt, [flat], one16)
            return carry

        lax.fori_loop(0, _CHR // 16, gbody, 0)

    def chunk_body(ci, carry):
        @pl.when(ci % 2 == 0)
        def _():
            pltpu.make_async_copy(xsrc(ci), xb0, s0).wait()
            process(xb0, ci)

            @pl.when(ci + 2 < _NCHUNK)
            def _():
                pltpu.async_copy(xsrc(ci + 2), xb0, s0)

        @pl.when(ci % 2 == 1)
        def _():
            pltpu.make_async_copy(xsrc(ci), xb1, s1).wait()
            process(xb1, ci)

            @pl.when(ci + 2 < _NCHUNK)
            def _():
                pltpu.async_copy(xsrc(ci + 2), xb1, s1)

        return carry

    lax.fori_loop(0, _NCHUNK, chunk_body, 0)

    for j in range(7):
        accs = zero16
        accc = zero16
        for l in range(16):
            accs = accs + binsum[pl.ds(l * 112 + j * 16, 16)]
            accc = accc + bincnt[pl.ds(l * 112 + j * 16, 16)]
        outv[0, pl.ds(j * 16, 16)] = accs
        outv[1, pl.ds(j * 16, 16)] = accc
    outv[0, pl.ds(112, 16)] = zero16
    outv[1, pl.ds(112, 16)] = zero16

    pltpu.sync_copy(outv.at[0], out_hbm.at[wid])
    pltpu.sync_copy(outv.at[1], out_hbm.at[_NW + wid])


_R = 8192
_GTC = (_B - _BSC) // _R


def _tc_body(x_ref, t_ref, w_ref, out_ref, acc_ref):
    i = pl.program_id(0)

    @pl.when(i == 0)
    def _():
        acc_ref[...] = jnp.zeros_like(acc_ref)

    x = x_ref[...]
    t = t_ref[...]
    w = w_ref[...]

    e = jnp.exp(x) * w
    cls = jax.lax.broadcasted_iota(jnp.int32, (_R, _NC), 1)
    e_masked = jnp.where(t == cls, e, 0.0)
    ones_row = jnp.ones((1, _NC), jnp.float32)
    s = jax.lax.dot_general(ones_row, e, (((1,), (1,)), ((), ())),
                            preferred_element_type=jnp.float32)
    et = jax.lax.dot_general(ones_row, e_masked, (((1,), (1,)), ((), ())),
                             preferred_element_type=jnp.float32)
    p = jnp.clip(et / s, 1e-5, 1.0)

    cls128 = jax.lax.broadcasted_iota(jnp.int32, (_R, 128), 1)
    oh128 = (t == cls128).astype(jnp.float32)
    pstack = jnp.concatenate([p, jnp.ones_like(p)], axis=0)
    part = jax.lax.dot_general(pstack, oh128, (((1,), (0,)), ((), ())),
                               preferred_element_type=jnp.float32)
    acc_ref[...] += part

    @pl.when(i == _GTC - 1)
    def _():
        out_ref[...] = acc_ref[...]


def _fin_body(scref, tcref, oref):
    sums = jnp.sum(scref[0:_NW, :], axis=0, keepdims=True) + tcref[0:1, :]
    counts = jnp.sum(scref[_NW:2 * _NW, :], axis=0, keepdims=True) \
        + tcref[1:2, :]
    exist = counts != 0.0
    denom = jnp.where(exist, counts, 1.0)
    meanp = sums / denom
    safe = jnp.where(exist, meanp, 1.0)
    ml = -jnp.log(safe)
    pw = jnp.where(exist, ml * ml * ml, 0.0)
    n_exist = jnp.sum(exist.astype(jnp.float32))
    msum = jnp.sum(pw) / n_exist
    loss = jnp.exp(jnp.log(msum) / 3.0)
    oref[...] = jnp.broadcast_to(loss, (1, 1))


def kernel(output, target, weight):
    mesh = plsc.VectorSubcoreMesh(core_axis_name="c", subcore_axis_name="s",
                                  num_cores=_NCORES, num_subcores=_NSUB)
    sc = pl.kernel(
        _sc_body,
        out_type=jax.ShapeDtypeStruct((2 * _NW, 128), jnp.float32),
        mesh=mesh,
        compiler_params=pltpu.CompilerParams(needs_layout_passes=False),
        scratch_types=[
            pltpu.VMEM((_CHR, _NC), jnp.float32),
            pltpu.VMEM((_CHR, _NC), jnp.float32),
            pltpu.VMEM((_RPW,), jnp.int32),
            pltpu.VMEM((112,), jnp.float32),
            pltpu.VMEM((1600,), jnp.float32),
            pltpu.VMEM((1792,), jnp.float32),
            pltpu.VMEM((1792,), jnp.float32),
            pltpu.VMEM((2, 128), jnp.float32),
            pltpu.SemaphoreType.DMA,
            pltpu.SemaphoreType.DMA,
            pltpu.SemaphoreType.DMA,
            pltpu.SemaphoreType.DMA,
        ],
    )
    sc_partials = sc(output, target, weight)
    t2 = target.reshape(_B, 1)
    tc_partials = pl.pallas_call(
        _tc_body,
        grid=(_GTC,),
        in_specs=[
            pl.BlockSpec((_R, _NC), lambda i: (i + _BSC // _R, 0)),
            pl.BlockSpec((_R, 1), lambda i: (i + _BSC // _R, 0)),
            pl.BlockSpec((1, _NC), lambda i: (0, 0)),
        ],
        out_specs=pl.BlockSpec((2, 128), lambda i: (0, 0)),
        out_shape=jax.ShapeDtypeStruct((2, 128), jnp.float32),
        scratch_shapes=[pltpu.VMEM((2, 128), jnp.float32)],
        compiler_params=pltpu.CompilerParams(
            dimension_semantics=("arbitrary",)),
    )(output, t2, weight.reshape(1, _NC))
    res = pl.pallas_call(
        _fin_body,
        out_shape=jax.ShapeDtypeStruct((1, 1), jnp.float32),
    )(sc_partials, tc_partials)
    return res[0, 0]


# speedup vs baseline: 2.2020x; 1.0125x over previous
import jax
import jax.numpy as jnp
from jax import lax
from jax.experimental import pallas as pl
from jax.experimental.pallas import tpu as pltpu
from jax.experimental.pallas import tpu_sc as plsc

_NC = 100
_B = 65536
_NCORES = 2
_NSUB = 16
_NW = _NCORES * _NSUB
_RPW = _B // _NW
_CHR = 256
_NCHUNK = _RPW // _CHR


def _sc_body(x_hbm, t_hbm, w_hbm, out_hbm,
             xb0, xb1, trow, wv, wrot, binsum, bincnt, outv,
             s0, s1, st, sw):
    wid = lax.axis_index("s") * _NCORES + lax.axis_index("c")
    base = wid * _RPW
    lane = lax.iota(jnp.int32, 16)
    lane112 = lane * 112
    zero16 = jnp.zeros((16,), jnp.float32)
    one16 = jnp.ones((16,), jnp.float32)

    tcp = pltpu.async_copy(t_hbm.at[pl.ds(base, _RPW)], trow, st)
    wv[pl.ds(96, 16)] = zero16
    wcp = pltpu.async_copy(w_hbm.at[pl.ds(0, _NC)], wv.at[pl.ds(0, _NC)], sw)

    def xsrc(ci):
        return x_hbm.at[pl.ds(base + ci * _CHR, _CHR), :]

    pltpu.async_copy(xsrc(0), xb0, s0)
    pltpu.async_copy(xsrc(1), xb1, s1)

    for l in range(16):
        for j in range(7):
            binsum[pl.ds(l * 112 + j * 16, 16)] = zero16
            bincnt[pl.ds(l * 112 + j * 16, 16)] = zero16

    tcp.wait()
    wcp.wait()

    def wrot_body(c, carry):
        cv = lane + c
        cv = jnp.where(cv >= _NC, cv - _NC, cv)
        wrot[pl.ds(c * 16, 16)] = plsc.load_gather(wv, [cv])
        return carry

    lax.fori_loop(0, _NC, wrot_body, 0)

    def process(xb, ci):
        def gbody(gq, carry):
            base_r = gq * 64
            rowidxs = [lane + (base_r + k * 16) for k in range(4)]
            saccs = [[zero16, zero16] for _ in range(4)]
            for c in range(_NC):
                cv = lane + c
                cv = jnp.where(cv >= _NC, cv - _NC, cv)
                wc = wrot[pl.ds(c * 16, 16)]
                for k in range(4):
                    col = plsc.load_gather(xb, [rowidxs[k], cv])
                    saccs[k][c % 2] = saccs[k][c % 2] + jnp.exp(col) * wc
            for k in range(4):
                sacc = saccs[k][0] + saccs[k][1]
                tg = trow[pl.ds(ci * _CHR + base_r + k * 16, 16)]
                xt = plsc.load_gather(xb, [rowidxs[k], tg])
                wt = plsc.load_gather(wv, [tg])
                et = jnp.exp(xt) * wt
                pr = jnp.minimum(jnp.maximum(et / sacc, 1e-5), 1.0)
                flat = lane112 + tg
                plsc.addupdate_scatter(binsum, [flat], pr)
                plsc.addupdate_scatter(bincnt, [flat], one16)
            return carry

        lax.fori_loop(0, _CHR // 64, gbody, 0)

    def chunk_body(ci, carry):
        @pl.when(ci % 2 == 0)
        def _():
            pltpu.make_async_copy(xsrc(ci), xb0, s0).wait()
            process(xb0, ci)

            @pl.when(ci + 2 < _NCHUNK)
            def _():
                pltpu.async_copy(xsrc(ci + 2), xb0, s0)

        @pl.when(ci % 2 == 1)
        def _():
            pltpu.make_async_copy(xsrc(ci), xb1, s1).wait()
            process(xb1, ci)

            @pl.when(ci + 2 < _NCHUNK)
            def _():
                pltpu.async_copy(xsrc(ci + 2), xb1, s1)

        return carry

    lax.fori_loop(0, _NCHUNK, chunk_body, 0)

    for j in range(7):
        accs = zero16
        accc = zero16
        for l in range(16):
            accs = accs + binsum[pl.ds(l * 112 + j * 16, 16)]
            accc = accc + bincnt[pl.ds(l * 112 + j * 16, 16)]
        outv[0, pl.ds(j * 16, 16)] = accs
        outv[1, pl.ds(j * 16, 16)] = accc
    outv[0, pl.ds(112, 16)] = zero16
    outv[1, pl.ds(112, 16)] = zero16

    pltpu.sync_copy(outv.at[0], out_hbm.at[wid])
    pltpu.sync_copy(outv.at[1], out_hbm.at[_NW + wid])


def _fin_body(pref, oref):
    sums = jnp.sum(pref[0:_NW, :], axis=0, keepdims=True)
    counts = jnp.sum(pref[_NW:2 * _NW, :], axis=0, keepdims=True)
    exist = counts != 0.0
    denom = jnp.where(exist, counts, 1.0)
    meanp = sums / denom
    safe = jnp.where(exist, meanp, 1.0)
    ml = -jnp.log(safe)
    pw = jnp.where(exist, ml * ml * ml, 0.0)
    n_exist = jnp.sum(exist.astype(jnp.float32))
    msum = jnp.sum(pw) / n_exist
    loss = jnp.exp(jnp.log(msum) / 3.0)
    oref[...] = jnp.broadcast_to(loss, (1, 1))


def kernel(output, target, weight):
    mesh = plsc.VectorSubcoreMesh(core_axis_name="c", subcore_axis_name="s",
                                  num_cores=_NCORES, num_subcores=_NSUB)
    sc = pl.kernel(
        _sc_body,
        out_type=jax.ShapeDtypeStruct((2 * _NW, 128), jnp.float32),
        mesh=mesh,
        compiler_params=pltpu.CompilerParams(needs_layout_passes=False),
        scratch_types=[
            pltpu.VMEM((_CHR, _NC), jnp.float32),
            pltpu.VMEM((_CHR, _NC), jnp.float32),
            pltpu.VMEM((_RPW,), jnp.int32),
            pltpu.VMEM((112,), jnp.float32),
            pltpu.VMEM((1600,), jnp.float32),
            pltpu.VMEM((1792,), jnp.float32),
            pltpu.VMEM((1792,), jnp.float32),
            pltpu.VMEM((2, 128), jnp.float32),
            pltpu.SemaphoreType.DMA,
            pltpu.SemaphoreType.DMA,
            pltpu.SemaphoreType.DMA,
            pltpu.SemaphoreType.DMA,
        ],
    )
    partials = sc(output, target, weight)
    res = pl.pallas_call(
        _fin_body,
        out_shape=jax.ShapeDtypeStruct((1, 1), jnp.float32),
    )(partials)
    return res[0, 0]
